# Initial kernel scaffold; baseline (speedup 1.0000x reference)
import jax
import jax.numpy as jnp
from jax.experimental import pallas as pl


def kernel(t, xs, params):
    raise NotImplementedError("write your pallas kernel here")


# speedup vs baseline: 24.3622x; 24.3622x over previous
import jax
import jax.numpy as jnp
from jax import lax
from jax.experimental import pallas as pl
from jax.experimental.pallas import tpu as pltpu

_B, _P, _D, _H = 128, 64, 3, 128
_NL = 4


def _egnn_body(t_ref, x_ref, embW_ref, embB_ref,
               e1Wa_ref, e1Wb_ref, e1wr_ref, e1we_ref, e1b_ref,
               e2W_ref, e2b_ref,
               c1W_ref, c1b_ref, c2W_ref,
               n1Wa_ref, n1Wb_ref, n1b_ref, n2W_ref, n2b_ref,
               out_ref):
    t = t_ref[0, 0]
    x = x_ref[0]

    h = t * embW_ref[...] + embB_ref[...]
    h = jnp.broadcast_to(h, (_P, _H))

    ii = lax.broadcasted_iota(jnp.int32, (_P, _P), 0)
    jj = lax.broadcasted_iota(jnp.int32, (_P, _P), 1)
    eyef = jnp.where(ii == jj, 1.0, 0.0)
    offdiag = 1.0 - eyef

    def gram_radial(c):
        g = lax.dot_general(c, c, (((1,), (1,)), ((), ())),
                            preferred_element_type=jnp.float32)
        diag = jnp.sum(g * eyef, axis=1, keepdims=True)
        diag_r = jnp.sum(g * eyef, axis=0, keepdims=True)
        return jnp.maximum(diag + diag_r - 2.0 * g, 0.0)

    ea = gram_radial(x)
    coord = x
    for l in range(_NL):
        radial = gram_radial(coord)
        rinv = 1.0 / (jnp.sqrt(radial + 1e-8) + 1.0)

        a = jnp.dot(h, e1Wa_ref[l], preferred_element_type=jnp.float32) + e1b_ref[l]
        b = jnp.dot(h, e1Wb_ref[l], preferred_element_type=jnp.float32)
        pre1 = (a[:, None, :] + b[None, :, :]
                + radial[:, :, None] * e1wr_ref[l][None, :, :]
                + ea[:, :, None] * e1we_ref[l][None, :, :])
        m1 = jax.nn.silu(pre1).reshape(_P * _P, _H)
        m = jax.nn.silu(jnp.dot(m1, e2W_ref[l],
                                preferred_element_type=jnp.float32) + e2b_ref[l])
        c1 = jax.nn.silu(jnp.dot(m, c1W_ref[l],
                                 preferred_element_type=jnp.float32) + c1b_ref[l])
        w = jnp.dot(c1, c2W_ref[l], preferred_element_type=jnp.float32)

        q = w.reshape(_P, _P, 1) * rinv[:, :, None]
        diff = coord[:, None, :] - coord[None, :, :]
        coord = coord + jnp.sum(q * diff, axis=1)

        if l < _NL - 1:
            agg = jnp.sum(m.reshape(_P, _P, _H) * offdiag[:, :, None], axis=1)
            u = jax.nn.silu(jnp.dot(h, n1Wa_ref[l], preferred_element_type=jnp.float32)
                            + jnp.dot(agg, n1Wb_ref[l], preferred_element_type=jnp.float32)
                            + n1b_ref[l])
            h = h + jnp.dot(u, n2W_ref[l], preferred_element_type=jnp.float32) + n2b_ref[l]

    vel = coord - x
    vel = vel - jnp.mean(vel, axis=0, keepdims=True)
    out_ref[...] = vel[None]


def _egnn_pallas(t, xs, params, interpret=False):
    x3 = xs.reshape(_B, _P, _D)
    layers = params["layers"]

    def stack(f):
        return jnp.stack([f(lp) for lp in layers])

    e1W = stack(lambda lp: lp["edge1"]["W"])
    e1Wa = e1W[:, :_H, :]
    e1Wb = e1W[:, _H:2 * _H, :]
    e1wr = e1W[:, 2 * _H:2 * _H + 1, :]
    e1we = e1W[:, 2 * _H + 1:2 * _H + 2, :]
    e1b = stack(lambda lp: lp["edge1"]["b"])[:, None, :]
    e2W = stack(lambda lp: lp["edge2"]["W"])
    e2b = stack(lambda lp: lp["edge2"]["b"])[:, None, :]
    c1W = stack(lambda lp: lp["coord1"]["W"])
    c1b = stack(lambda lp: lp["coord1"]["b"])[:, None, :]
    c2W = stack(lambda lp: lp["coord2"]["W"])
    n1W = stack(lambda lp: lp["node1"]["W"])
    n1Wa = n1W[:, :_H, :]
    n1Wb = n1W[:, _H:, :]
    n1b = stack(lambda lp: lp["node1"]["b"])[:, None, :]
    n2W = stack(lambda lp: lp["node2"]["W"])
    n2b = stack(lambda lp: lp["node2"]["b"])[:, None, :]

    t2 = t.reshape(1, 1).astype(jnp.float32)
    embW = params["embedding"]["W"]
    embB = params["embedding"]["b"][None, :]

    ops = [t2, x3, embW, embB, e1Wa, e1Wb, e1wr, e1we, e1b,
           e2W, e2b, c1W, c1b, c2W, n1Wa, n1Wb, n1b, n2W, n2b]

    def bcast_spec(arr):
        nd = arr.ndim
        return pl.BlockSpec(arr.shape, lambda g, _n=nd: (0,) * _n)

    in_specs = [pl.BlockSpec((1, 1), lambda g: (0, 0), memory_space=pltpu.SMEM),
                pl.BlockSpec((1, _P, _D), lambda g: (g, 0, 0))]
    in_specs += [bcast_spec(a) for a in ops[2:]]

    out = pl.pallas_call(
        _egnn_body,
        grid=(_B,),
        in_specs=in_specs,
        out_specs=pl.BlockSpec((1, _P, _D), lambda g: (g, 0, 0)),
        out_shape=jax.ShapeDtypeStruct((_B, _P, _D), jnp.float32),
        interpret=interpret,
    )(*ops)
    return out.reshape(_B, _P * _D)


def kernel(t, xs, params):
    return _egnn_pallas(t, xs, params)
